# baseline (speedup 1.0000x reference)
import jax
import jax.numpy as jnp
import numpy as np
from jax.experimental import pallas as pl
from jax.experimental.pallas import tpu as pltpu

_TOPK = 512
_BM = 128

_INT_MIN = np.int32(-(2**31))


def _fused_kernel(xh_ref, xl_ref, wh_hbm, wl_hbm, b_ref, o_ref,
                  wh_s, wl_s, y_s, sem_h, sem_l):
    i = pl.program_id(0)
    nsteps = pl.num_programs(0)

    @pl.when(i == 0)
    def _load_w():
        cp_h = pltpu.make_async_copy(wh_hbm, wh_s, sem_h)
        cp_l = pltpu.make_async_copy(wl_hbm, wl_s, sem_l)
        cp_h.start()
        cp_l.start()
        cp_h.wait()
        cp_l.wait()

    @pl.when(i > 0)
    def _epilogue():
        y = y_s[...]
        i32 = jax.lax.bitcast_convert_type(y, jnp.int32)
        v = jnp.where(i32 >= 0, i32, i32 ^ np.int32(0x7FFFFFFF))
        tb = jnp.zeros((v.shape[0], 1), jnp.int32)
        for j in range(31, -1, -1):
            bit = np.uint32(1 << j).view(np.int32)
            cand = tb | bit
            cnt = jnp.sum((v >= (cand ^ _INT_MIN)).astype(jnp.int32),
                          axis=1, keepdims=True)
            tb = jnp.where(cnt >= _TOPK, cand, tb)
        thr = tb ^ _INT_MIN
        o_ref[...] = jnp.where(v >= thr, y, 0.0)

    @pl.when(i < nsteps - 1)
    def _matmul():
        dims = (((1,), (0,)), ((), ()))
        xh = xh_ref[...]
        xl = xl_ref[...]
        acc = jax.lax.dot_general(xh, wh_s[...], dims,
                                  preferred_element_type=jnp.float32)
        acc = acc + jax.lax.dot_general(xh, wl_s[...], dims,
                                        preferred_element_type=jnp.float32)
        acc = acc + jax.lax.dot_general(xl, wh_s[...], dims,
                                        preferred_element_type=jnp.float32)
        y = acc + b_ref[...]
        y_s[...] = jnp.where(y >= 0.0, y, 0.1 * y)


def kernel(input, W, b):
    m, k = input.shape
    n = W.shape[0]
    xh = input.astype(jnp.bfloat16)
    xl = (input - xh.astype(jnp.float32)).astype(jnp.bfloat16)
    wh = W.astype(jnp.bfloat16)
    wl = (W - wh.astype(jnp.float32)).astype(jnp.bfloat16)
    wht = wh.T
    wlt = wl.T
    b2 = b.reshape(1, n)

    nblk = m // _BM
    grid = (nblk + 1,)
    x_map = lambda i: (jnp.minimum(i, nblk - 1), 0)
    o_map = lambda i: (jnp.maximum(i - 1, 0), 0)
    return pl.pallas_call(
        _fused_kernel,
        grid=grid,
        in_specs=[
            pl.BlockSpec((_BM, k), x_map),
            pl.BlockSpec((_BM, k), x_map),
            pl.BlockSpec(memory_space=pl.ANY),
            pl.BlockSpec(memory_space=pl.ANY),
            pl.BlockSpec((1, n), lambda i: (0, 0)),
        ],
        out_specs=pl.BlockSpec((_BM, n), o_map),
        out_shape=jax.ShapeDtypeStruct((m, n), jnp.float32),
        scratch_shapes=[
            pltpu.VMEM((k, n), jnp.bfloat16),
            pltpu.VMEM((k, n), jnp.bfloat16),
            pltpu.VMEM((_BM, n), jnp.float32),
            pltpu.SemaphoreType.DMA,
            pltpu.SemaphoreType.DMA,
        ],
        compiler_params=pltpu.CompilerParams(
            dimension_semantics=("arbitrary",),
        ),
    )(xh, xl, wht, wlt, b2)


# speedup vs baseline: 1.0014x; 1.0014x over previous
import jax
import jax.numpy as jnp
import numpy as np
from jax.experimental import pallas as pl
from jax.experimental.pallas import tpu as pltpu

_TOPK = 512
_BM = 128

_INT_MIN = np.int32(-(2**31))


def _fused_kernel(xh_ref, xl_ref, wh_hbm, wl_hbm, b_ref, o_ref,
                  wh_s, wl_s, y_s, sem_h, sem_l):
    i = pl.program_id(0)
    nsteps = pl.num_programs(0)

    @pl.when(i == 0)
    def _load_w():
        cp_h = pltpu.make_async_copy(wh_hbm, wh_s, sem_h)
        cp_l = pltpu.make_async_copy(wl_hbm, wl_s, sem_l)
        cp_h.start()
        cp_l.start()
        cp_h.wait()
        cp_l.wait()

    y = y_s[...]
    i32 = jax.lax.bitcast_convert_type(y, jnp.int32)
    v = jnp.where(i32 >= 0, i32, i32 ^ np.int32(0x7FFFFFFF))
    tb = jnp.zeros((v.shape[0], 1), jnp.int32)
    for j in range(31, -1, -1):
        bit = np.uint32(1 << j).view(np.int32)
        cand = tb | bit
        cnt = jnp.sum((v >= (cand ^ _INT_MIN)).astype(jnp.int32),
                      axis=1, keepdims=True)
        tb = jnp.where(cnt >= _TOPK, cand, tb)
    thr = tb ^ _INT_MIN
    o_ref[...] = jnp.where(v >= thr, y, 0.0)

    dims = (((1,), (0,)), ((), ()))
    xh = xh_ref[...]
    xl = xl_ref[...]
    acc = jax.lax.dot_general(xh, wh_s[...], dims,
                              preferred_element_type=jnp.float32)
    acc = acc + jax.lax.dot_general(xh, wl_s[...], dims,
                                    preferred_element_type=jnp.float32)
    acc = acc + jax.lax.dot_general(xl, wh_s[...], dims,
                                    preferred_element_type=jnp.float32)
    ym = acc + b_ref[...]
    y_s[...] = jnp.where(ym >= 0.0, ym, 0.1 * ym)


def kernel(input, W, b):
    m, k = input.shape
    n = W.shape[0]
    xh = input.astype(jnp.bfloat16)
    xl = (input - xh.astype(jnp.float32)).astype(jnp.bfloat16)
    wh = W.astype(jnp.bfloat16)
    wl = (W - wh.astype(jnp.float32)).astype(jnp.bfloat16)
    wht = wh.T
    wlt = wl.T
    b2 = b.reshape(1, n)

    nblk = m // _BM
    grid = (nblk + 1,)
    x_map = lambda i: (jnp.minimum(i, nblk - 1), 0)
    o_map = lambda i: (jnp.maximum(i - 1, 0), 0)
    return pl.pallas_call(
        _fused_kernel,
        grid=grid,
        in_specs=[
            pl.BlockSpec((_BM, k), x_map),
            pl.BlockSpec((_BM, k), x_map),
            pl.BlockSpec(memory_space=pl.ANY),
            pl.BlockSpec(memory_space=pl.ANY),
            pl.BlockSpec((1, n), lambda i: (0, 0)),
        ],
        out_specs=pl.BlockSpec((_BM, n), o_map),
        out_shape=jax.ShapeDtypeStruct((m, n), jnp.float32),
        scratch_shapes=[
            pltpu.VMEM((k, n), jnp.bfloat16),
            pltpu.VMEM((k, n), jnp.bfloat16),
            pltpu.VMEM((_BM, n), jnp.float32),
            pltpu.SemaphoreType.DMA,
            pltpu.SemaphoreType.DMA,
        ],
        compiler_params=pltpu.CompilerParams(
            dimension_semantics=("arbitrary",),
        ),
    )(xh, xl, wht, wlt, b2)


# speedup vs baseline: 1.0297x; 1.0283x over previous
import jax
import jax.numpy as jnp
import numpy as np
from jax.experimental import pallas as pl
from jax.experimental.pallas import tpu as pltpu

_TOPK = 512
_BM = 128

_INT_MIN = np.int32(-(2**31))


def _fused_kernel(x_ref, wh_hbm, wl_hbm, b_ref, o_ref,
                  wh_s, wl_s, y_s, sem_h, sem_l):
    i = pl.program_id(0)

    @pl.when(i == 0)
    def _load_w():
        cp_h = pltpu.make_async_copy(wh_hbm, wh_s, sem_h)
        cp_l = pltpu.make_async_copy(wl_hbm, wl_s, sem_l)
        cp_h.start()
        cp_l.start()
        cp_h.wait()
        cp_l.wait()

    y = y_s[...]
    i32 = jax.lax.bitcast_convert_type(y, jnp.int32)
    v = jnp.where(i32 >= 0, i32, i32 ^ np.int32(0x7FFFFFFF))
    tb = jnp.zeros((v.shape[0], 1), jnp.int32)
    for j in range(31, -1, -1):
        bit = np.uint32(1 << j).view(np.int32)
        cand = tb | bit
        cnt = jnp.sum((v >= (cand ^ _INT_MIN)).astype(jnp.int32),
                      axis=1, keepdims=True)
        tb = jnp.where(cnt >= _TOPK, cand, tb)
    thr = tb ^ _INT_MIN
    o_ref[...] = jnp.where(v >= thr, y, 0.0)

    x = x_ref[...]
    xh = x.astype(jnp.bfloat16)
    xl = (x - xh.astype(jnp.float32)).astype(jnp.bfloat16)
    dims = (((1,), (0,)), ((), ()))
    acc = jax.lax.dot_general(xh, wh_s[...], dims,
                              preferred_element_type=jnp.float32)
    acc = acc + jax.lax.dot_general(xh, wl_s[...], dims,
                                    preferred_element_type=jnp.float32)
    acc = acc + jax.lax.dot_general(xl, wh_s[...], dims,
                                    preferred_element_type=jnp.float32)
    ym = acc + b_ref[...]
    y_s[...] = jnp.where(ym >= 0.0, ym, 0.1 * ym)


def kernel(input, W, b):
    m, k = input.shape
    n = W.shape[0]
    wh = W.astype(jnp.bfloat16)
    wl = (W - wh.astype(jnp.float32)).astype(jnp.bfloat16)
    wht = wh.T
    wlt = wl.T
    b2 = b.reshape(1, n)

    nblk = m // _BM
    grid = (nblk + 1,)
    x_map = lambda i: (jnp.minimum(i, nblk - 1), 0)
    o_map = lambda i: (jnp.maximum(i - 1, 0), 0)
    return pl.pallas_call(
        _fused_kernel,
        grid=grid,
        in_specs=[
            pl.BlockSpec((_BM, k), x_map),
            pl.BlockSpec(memory_space=pl.ANY),
            pl.BlockSpec(memory_space=pl.ANY),
            pl.BlockSpec((1, n), lambda i: (0, 0)),
        ],
        out_specs=pl.BlockSpec((_BM, n), o_map),
        out_shape=jax.ShapeDtypeStruct((m, n), jnp.float32),
        scratch_shapes=[
            pltpu.VMEM((k, n), jnp.bfloat16),
            pltpu.VMEM((k, n), jnp.bfloat16),
            pltpu.VMEM((_BM, n), jnp.float32),
            pltpu.SemaphoreType.DMA,
            pltpu.SemaphoreType.DMA,
        ],
        compiler_params=pltpu.CompilerParams(
            dimension_semantics=("arbitrary",),
        ),
    )(input, wht, wlt, b2)


# speedup vs baseline: 1.1694x; 1.1356x over previous
import jax
import jax.numpy as jnp
import numpy as np
from jax.experimental import pallas as pl
from jax.experimental.pallas import tpu as pltpu

_TOPK = 512
_BM = 128

_INT_MIN = np.int32(-(2**31))


def _fused_kernel(xh_ref, xl_ref, wh_hbm, wl_hbm, b_ref, o_ref,
                  wh_s, wl_s, sem_h, sem_l):
    @pl.when(pl.program_id(0) == 0)
    def _load_w():
        cp_h = pltpu.make_async_copy(wh_hbm, wh_s, sem_h)
        cp_l = pltpu.make_async_copy(wl_hbm, wl_s, sem_l)
        cp_h.start()
        cp_l.start()
        cp_h.wait()
        cp_l.wait()

    dims = (((1,), (0,)), ((), ()))
    xh = xh_ref[...]
    xl = xl_ref[...]
    acc = jax.lax.dot_general(xh, wh_s[...], dims,
                              preferred_element_type=jnp.float32)
    acc = acc + jax.lax.dot_general(xh, wl_s[...], dims,
                                    preferred_element_type=jnp.float32)
    acc = acc + jax.lax.dot_general(xl, wh_s[...], dims,
                                    preferred_element_type=jnp.float32)
    y = acc + b_ref[...]
    y = jnp.where(y >= 0.0, y, 0.1 * y)

    i32 = jax.lax.bitcast_convert_type(y, jnp.int32)
    v = jnp.where(i32 >= 0, i32, i32 ^ np.int32(0x7FFFFFFF))

    tb = jnp.zeros((v.shape[0], 1), jnp.int32)
    for j in range(31, -1, -1):
        bit = np.uint32(1 << j).view(np.int32)
        cand = tb | bit
        cnt = jnp.sum((v >= (cand ^ _INT_MIN)).astype(jnp.int32),
                      axis=1, keepdims=True)
        tb = jnp.where(cnt >= _TOPK, cand, tb)
    thr = tb ^ _INT_MIN
    o_ref[...] = jnp.where(v >= thr, y, 0.0)


def kernel(input, W, b):
    m, k = input.shape
    n = W.shape[0]
    xh = input.astype(jnp.bfloat16)
    xl = (input - xh.astype(jnp.float32)).astype(jnp.bfloat16)
    wh = W.astype(jnp.bfloat16)
    wl = (W - wh.astype(jnp.float32)).astype(jnp.bfloat16)
    wht = wh.T
    wlt = wl.T
    b2 = b.reshape(1, n)

    grid = (m // _BM,)
    return pl.pallas_call(
        _fused_kernel,
        grid=grid,
        in_specs=[
            pl.BlockSpec((_BM, k), lambda i: (i, 0)),
            pl.BlockSpec((_BM, k), lambda i: (i, 0)),
            pl.BlockSpec(memory_space=pl.ANY),
            pl.BlockSpec(memory_space=pl.ANY),
            pl.BlockSpec((1, n), lambda i: (0, 0)),
        ],
        out_specs=pl.BlockSpec((_BM, n), lambda i: (i, 0)),
        out_shape=jax.ShapeDtypeStruct((m, n), jnp.float32),
        scratch_shapes=[
            pltpu.VMEM((k, n), jnp.bfloat16),
            pltpu.VMEM((k, n), jnp.bfloat16),
            pltpu.SemaphoreType.DMA,
            pltpu.SemaphoreType.DMA,
        ],
        compiler_params=pltpu.CompilerParams(
            dimension_semantics=("parallel",),
        ),
    )(xh, xl, wht, wlt, b2)
